# baseline (speedup 1.0000x reference)
import jax
import jax.numpy as jnp
from jax import lax
from jax.experimental import pallas as pl
from jax.experimental.pallas import tpu as pltpu
from jax.experimental.pallas import tpu_sc as plsc

NC, NS = 2, 16
NW = NC * NS


def sc_add(hidden_states, idx, control_vectors, M, K, CH=8):
    B, S, E = hidden_states.shape
    n = control_vectors.shape[0]
    wpb = NW // M
    rpw = S // wpb
    nch = rpw // CH
    EV = E // 16

    def body(h_hbm, idx_hbm, cv_hbm, out_hbm,
             priv_idx, myrow_v, in0, in1, out0, out1,
             gsem, isem0, isem1, osem0, osem1):
        wid = lax.axis_index("s") * NC + lax.axis_index("c")
        boff = wid // wpb
        row0 = (wid % wpb) * rpw
        batch = K + boff

        pltpu.sync_copy(idx_hbm.at[batch], priv_idx)
        pltpu.async_copy(
            cv_hbm.at[priv_idx.at[pl.ds(0, 1)]], myrow_v, gsem
        ).wait()

        ins = (in0, in1)
        outs = (out0, out1)
        isems = (isem0, isem1)
        osems = (osem0, osem1)

        def start_in(j, chunk):
            pltpu.make_async_copy(
                h_hbm.at[batch, pl.ds(row0 + chunk * CH, CH)], ins[j], isems[j]
            ).start()

        def wait_in(j):
            pltpu.make_async_copy(
                h_hbm.at[batch, pl.ds(row0, CH)], ins[j], isems[j]
            ).wait()

        def start_out(j, chunk):
            pltpu.make_async_copy(
                outs[j], out_hbm.at[boff, pl.ds(row0 + chunk * CH, CH)], osems[j]
            ).start()

        def wait_out(j):
            pltpu.make_async_copy(
                outs[j], out_hbm.at[boff, pl.ds(row0, CH)], osems[j]
            ).wait()

        def compute(j):
            @pl.loop(0, EV, unroll=8)
            def _(e):
                cv = myrow_v[0, pl.ds(e * 16, 16)]
                for s in range(CH):
                    outs[j][s, pl.ds(e * 16, 16)] = (
                        ins[j][s, pl.ds(e * 16, 16)] + cv
                    )

        start_in(0, 0)
        start_in(1, 1)

        @pl.loop(0, nch, step=2)
        def _(i):
            for j in range(2):
                ii = i + j
                wait_in(j)

                @pl.when(ii >= 2)
                def _():
                    wait_out(j)

                compute(j)
                start_out(j, ii)

                @pl.when(ii + 2 < nch)
                def _():
                    start_in(j, ii + 2)

        wait_out(0)
        wait_out(1)

    mesh = plsc.VectorSubcoreMesh(core_axis_name="c", subcore_axis_name="s")
    return pl.kernel(
        body,
        out_type=jax.ShapeDtypeStruct((M, S, E), jnp.float32),
        mesh=mesh,
        scratch_types=[
            pltpu.VMEM((16,), jnp.int32),
            pltpu.VMEM((1, E), jnp.float32),
            pltpu.VMEM((CH, E), jnp.float32),
            pltpu.VMEM((CH, E), jnp.float32),
            pltpu.VMEM((CH, E), jnp.float32),
            pltpu.VMEM((CH, E), jnp.float32),
            pltpu.SemaphoreType.DMA,
            pltpu.SemaphoreType.DMA,
            pltpu.SemaphoreType.DMA,
            pltpu.SemaphoreType.DMA,
            pltpu.SemaphoreType.DMA,
        ],
    )(hidden_states, idx, control_vectors)


def kernel(hidden_states, affective_state_indices, control_vectors):
    B, S, E = hidden_states.shape
    n = control_vectors.shape[0]
    idx = jnp.clip(affective_state_indices.astype(jnp.int32), 0, n - 1)
    idx_rep = jnp.broadcast_to(idx[:, None], (B, 16))
    return sc_add(hidden_states, idx_rep, control_vectors, M=B, K=0)


# speedup vs baseline: 1.7918x; 1.7918x over previous
import jax
import jax.numpy as jnp
from jax import lax
from jax.experimental import pallas as pl
from jax.experimental.pallas import tpu as pltpu
from jax.experimental.pallas import tpu_sc as plsc

NC, NS = 2, 16
NW = NC * NS
K_TC = 28


def sc_add(hidden_states, idx_rep, control_vectors, M, K, CH=8):
    B, S, E = hidden_states.shape
    wpb = NW // M
    rpw = S // wpb
    nch = rpw // CH
    EV = E // 16

    def body(h_hbm, idx_hbm, cv_hbm, out_hbm,
             priv_idx, myrow_v, in0, in1, out0, out1,
             gsem, isem0, isem1, osem0, osem1):
        wid = lax.axis_index("s") * NC + lax.axis_index("c")
        boff = wid // wpb
        row0 = (wid % wpb) * rpw
        batch = K + boff

        pltpu.sync_copy(idx_hbm.at[batch], priv_idx)
        pltpu.async_copy(
            cv_hbm.at[priv_idx.at[pl.ds(0, 1)]], myrow_v, gsem
        ).wait()

        ins = (in0, in1)
        outs = (out0, out1)
        isems = (isem0, isem1)
        osems = (osem0, osem1)

        def start_in(j, chunk):
            pltpu.make_async_copy(
                h_hbm.at[batch, pl.ds(row0 + chunk * CH, CH)], ins[j], isems[j]
            ).start()

        def wait_in(j):
            pltpu.make_async_copy(
                h_hbm.at[batch, pl.ds(row0, CH)], ins[j], isems[j]
            ).wait()

        def start_out(j, chunk):
            pltpu.make_async_copy(
                outs[j], out_hbm.at[boff, pl.ds(row0 + chunk * CH, CH)], osems[j]
            ).start()

        def wait_out(j):
            pltpu.make_async_copy(
                outs[j], out_hbm.at[boff, pl.ds(row0, CH)], osems[j]
            ).wait()

        def compute(j):
            @pl.loop(0, EV, unroll=8)
            def _(e):
                cv = myrow_v[0, pl.ds(e * 16, 16)]
                for s in range(CH):
                    outs[j][s, pl.ds(e * 16, 16)] = (
                        ins[j][s, pl.ds(e * 16, 16)] + cv
                    )

        start_in(0, 0)
        start_in(1, 1)

        @pl.loop(0, nch, step=2)
        def _(i):
            for j in range(2):
                ii = i + j
                wait_in(j)

                @pl.when(ii >= 2)
                def _():
                    wait_out(j)

                compute(j)
                start_out(j, ii)

                @pl.when(ii + 2 < nch)
                def _():
                    start_in(j, ii + 2)

        wait_out(0)
        wait_out(1)

    mesh = plsc.VectorSubcoreMesh(core_axis_name="c", subcore_axis_name="s")
    return pl.kernel(
        body,
        out_type=jax.ShapeDtypeStruct((M, S, E), jnp.float32),
        mesh=mesh,
        scratch_types=[
            pltpu.VMEM((16,), jnp.int32),
            pltpu.VMEM((1, E), jnp.float32),
            pltpu.VMEM((CH, E), jnp.float32),
            pltpu.VMEM((CH, E), jnp.float32),
            pltpu.VMEM((CH, E), jnp.float32),
            pltpu.VMEM((CH, E), jnp.float32),
            pltpu.SemaphoreType.DMA,
            pltpu.SemaphoreType.DMA,
            pltpu.SemaphoreType.DMA,
            pltpu.SemaphoreType.DMA,
            pltpu.SemaphoreType.DMA,
        ],
    )(hidden_states, idx_rep, control_vectors)


def _tc_body(idx_ref, h_ref, cv_ref, o_ref):
    o_ref[...] = h_ref[...] + cv_ref[0]


def tc_add(hidden_states, idx, control_vectors, K):
    B, S, E = hidden_states.shape
    n = control_vectors.shape[0]
    cv3 = control_vectors.reshape(n, 1, E)

    def h_map(b, s, idx_ref):
        return (b, s, 0)

    def cv_map(b, s, idx_ref):
        return (jnp.clip(idx_ref[b], 0, n - 1), 0, 0)

    return pl.pallas_call(
        _tc_body,
        grid_spec=pltpu.PrefetchScalarGridSpec(
            num_scalar_prefetch=1,
            grid=(K, 1),
            in_specs=[
                pl.BlockSpec((1, S, E), h_map),
                pl.BlockSpec((1, 1, E), cv_map),
            ],
            out_specs=pl.BlockSpec((1, S, E), h_map),
        ),
        out_shape=jax.ShapeDtypeStruct((K, S, E), hidden_states.dtype),
    )(idx, hidden_states, cv3)


def kernel(hidden_states, affective_state_indices, control_vectors):
    B, S, E = hidden_states.shape
    n = control_vectors.shape[0]
    idx = jnp.clip(affective_state_indices.astype(jnp.int32), 0, n - 1)
    idx_rep = jnp.broadcast_to(idx[:, None], (B, 16))
    M = B - K_TC
    sc_out = sc_add(hidden_states, idx_rep, control_vectors, M=M, K=K_TC)
    tc_out = tc_add(hidden_states, idx, control_vectors, K=K_TC)
    return jnp.concatenate([tc_out, sc_out], axis=0)


# speedup vs baseline: 3.7271x; 2.0801x over previous
import jax
import jax.numpy as jnp
from jax.experimental import pallas as pl
from jax.experimental.pallas import tpu as pltpu


def _body(idx_ref, h_ref, cv_ref, o_ref):
    o_ref[...] = h_ref[...] + cv_ref[0]


def kernel(hidden_states, affective_state_indices, control_vectors):
    B, S, E = hidden_states.shape
    n = control_vectors.shape[0]
    idx = affective_state_indices.astype(jnp.int32)
    cv3 = control_vectors.reshape(n, 1, E)
    BS = 1024
    grid = (B, S // BS)

    def h_map(b, s, idx_ref):
        return (b, s, 0)

    def cv_map(b, s, idx_ref):
        return (jnp.clip(idx_ref[b], 0, n - 1), 0, 0)

    return pl.pallas_call(
        _body,
        grid_spec=pltpu.PrefetchScalarGridSpec(
            num_scalar_prefetch=1,
            grid=grid,
            in_specs=[
                pl.BlockSpec((1, BS, E), h_map),
                pl.BlockSpec((1, 1, E), cv_map),
            ],
            out_specs=pl.BlockSpec((1, BS, E), h_map),
        ),
        out_shape=jax.ShapeDtypeStruct((B, S, E), hidden_states.dtype),
    )(idx, hidden_states, cv3)
